# Initial kernel scaffold; baseline (speedup 1.0000x reference)
import jax
import jax.numpy as jnp
from jax.experimental import pallas as pl


def kernel(token_ids, embedding_table):
    raise NotImplementedError("write your pallas kernel here")


# speedup vs baseline: 3.5754x; 3.5754x over previous
import functools

import jax
import jax.numpy as jnp
from jax import lax
from jax.experimental import pallas as pl
from jax.experimental.pallas import tpu as pltpu
from jax.experimental.pallas import tpu_sc as plsc

NC = 2
NS = 16
NW = NC * NS

IDX_ROW = 128
ROWS_PER_CHUNK = 1024
K = ROWS_PER_CHUNK // IDX_ROW


def _sc_gather(table, ids3, n_rows_per_w, d):
    n_idx_rows = ids3.shape[1]
    n_chunks = n_rows_per_w // ROWS_PER_CHUNK
    mesh = plsc.VectorSubcoreMesh(
        core_axis_name="c", subcore_axis_name="s", num_cores=NC,
        num_subcores=NS)

    @functools.partial(
        pl.kernel,
        mesh=mesh,
        compiler_params=pltpu.CompilerParams(use_tc_tiling_on_sc=False),
        out_type=jax.ShapeDtypeStruct((NW * n_rows_per_w, d), jnp.float32),
        scratch_types=[
            pltpu.VMEM((n_idx_rows, IDX_ROW), jnp.int32),
            pltpu.VMEM((ROWS_PER_CHUNK, d), jnp.float32),
            pltpu.SemaphoreType.DMA,
        ],
    )
    def k(table_hbm, idx_hbm, out_hbm, idx_v, rows_v, gsem):
        wid = lax.axis_index("s") * NC + lax.axis_index("c")
        pltpu.sync_copy(idx_hbm.at[wid], idx_v)
        base = wid * n_rows_per_w

        def chunk_body(c, carry):
            copies = []
            for j in range(K):
                copies.append(
                    pltpu.async_copy(
                        table_hbm.at[idx_v.at[c * K + j]],
                        rows_v.at[pl.ds(j * IDX_ROW, IDX_ROW)],
                        gsem,
                    ))
            for cp in copies:
                cp.wait()
            pltpu.sync_copy(
                rows_v, out_hbm.at[pl.ds(base + c * ROWS_PER_CHUNK,
                                         ROWS_PER_CHUNK)])
            return carry

        lax.fori_loop(0, n_chunks, chunk_body, 0)

    return k(table, ids3)


def kernel(token_ids, embedding_table):
    b, s = token_ids.shape
    v, d = embedding_table.shape
    n = b * s
    assert n % (NW * ROWS_PER_CHUNK) == 0
    n_rows_per_w = n // NW
    ids3 = token_ids.reshape(NW, n_rows_per_w // IDX_ROW, IDX_ROW)
    ids3 = ids3.astype(jnp.int32)
    out = _sc_gather(embedding_table, ids3, n_rows_per_w, d)
    return out.reshape(b, s, d)
